# baseline (speedup 1.0000x reference)
import functools

import jax
import jax.numpy as jnp
from jax import lax
from jax.experimental import pallas as pl
from jax.experimental.pallas import tpu as pltpu
from jax.experimental.pallas import tpu_sc as plsc

N = 10000
E = 320000
H = 128
T = 4

NC = 2
NS = 16
NW = NC * NS
EH = E
EW = EH // NW
GC = 80
NCHUNK = EW // GC
NROWS = 624
NREM = N - NS * NROWS

_PERM = []
for _g in range(4):
    for _i in range(16):
        _PERM.append(_g * 32 + _i)
        _PERM.append(_g * 32 + 16 + _i)


def _k1_body(x_ref, wsrc_ref, wdst_ref, b1_ref, y1_ref, y2_ref):
    x = x_ref[...]
    y1_ref[...] = jnp.dot(x, wsrc_ref[...], preferred_element_type=jnp.float32)
    y2_ref[...] = (
        jnp.dot(x, wdst_ref[...], preferred_element_type=jnp.float32)
        + b1_ref[...]
    )


def _precompute_y(x, wsrc, wdst, b1flat):
    blk = 1000
    return pl.pallas_call(
        _k1_body,
        grid=(N // blk,),
        in_specs=[
            pl.BlockSpec((blk, H), lambda i: (i, 0)),
            pl.BlockSpec((H, T * H), lambda i: (0, 0)),
            pl.BlockSpec((H, T * H), lambda i: (0, 0)),
            pl.BlockSpec((1, T * H), lambda i: (0, 0)),
        ],
        out_specs=[
            pl.BlockSpec((blk, T * H), lambda i: (i, 0)),
            pl.BlockSpec((blk, T * H), lambda i: (i, 0)),
        ],
        out_shape=[
            jax.ShapeDtypeStruct((N, T * H), jnp.float32),
            jax.ShapeDtypeStruct((N, T * H), jnp.float32),
        ],
    )(x, wsrc, wdst, b1flat)


def _k1b_body(src_ref, dst_ref, et_ref, i1_ref, i2_ref):
    et = et_ref[...]
    i1_ref[...] = src_ref[...] * T + et
    i2_ref[...] = dst_ref[...] * T + et


def _edge_indices(src2d, dst2d, et2d):
    rows = E // H
    return pl.pallas_call(
        _k1b_body,
        grid=(1,),
        in_specs=[pl.BlockSpec((rows, H), lambda i: (0, 0))] * 3,
        out_specs=[pl.BlockSpec((rows, H), lambda i: (0, 0))] * 2,
        out_shape=[jax.ShapeDtypeStruct((rows, H), jnp.int32)] * 2,
    )(src2d, dst2d, et2d)


def _sc_gather_body(y1_hbm, y2_hbm, i1_hbm, i2_hbm, hm_hbm,
                    i1_all, i2_all, abuf, bbuf, obuf, sa, sb, so):
    c = lax.axis_index("c")
    s = lax.axis_index("s")
    wid = s * NC + c
    gbase = wid * EW
    base0 = wid * EW

    pltpu.sync_copy(i1_hbm.at[pl.ds(gbase, EW)], i1_all)
    pltpu.sync_copy(i2_hbm.at[pl.ds(gbase, EW)], i2_all)

    def issue(k, b):
        off = k * GC
        pltpu.async_copy(y1_hbm.at[i1_all.at[pl.ds(off, GC)]], abuf[b], sa[b])
        pltpu.async_copy(y2_hbm.at[i2_all.at[pl.ds(off, GC)]], bbuf[b], sb[b])

    issue(0, 0)
    issue(1, 1)

    def step(k, b):
        off = k * GC
        pltpu.make_async_copy(y1_hbm.at[i1_all.at[pl.ds(off, GC)]],
                              abuf[b], sa[b]).wait()
        pltpu.make_async_copy(y2_hbm.at[i2_all.at[pl.ds(off, GC)]],
                              bbuf[b], sb[b]).wait()

        @pl.when(k >= 2)
        def _():
            pltpu.make_async_copy(
                obuf[b], hm_hbm.at[pl.ds(base0 + (k - 2) * GC, GC)],
                so[b]).wait()

        def rows(r, carry):
            for g in range(H // 32):
                sl0 = pl.ds(g * 32, 16)
                sl1 = pl.ds(g * 32 + 16, 16)
                s0 = jnp.maximum(abuf[b][r, sl0] + bbuf[b][r, sl0], 0.0)
                s1 = jnp.maximum(abuf[b][r, sl1] + bbuf[b][r, sl1], 0.0)
                p = plsc.pack(s0, s1, format=plsc.PackFormat.INTERLEAVED)
                obuf[b][r, pl.ds(g * 16, 16)] = plsc.bitcast(p, jnp.int32)
            return carry

        lax.fori_loop(0, GC, rows, 0)

        @pl.when(k < NCHUNK - 2)
        def _():
            issue(k + 2, b)

        pltpu.async_copy(obuf[b], hm_hbm.at[pl.ds(base0 + off, GC)], so[b])

    def chunk(k, carry):
        @pl.when(k % 2 == 0)
        def _():
            step(k, 0)

        @pl.when(k % 2 == 1)
        def _():
            step(k, 1)

        return carry

    lax.fori_loop(0, NCHUNK, chunk, 0)

    for b, k in ((0, NCHUNK - 1), (1, NCHUNK - 2)):
        pltpu.make_async_copy(
            obuf[b], hm_hbm.at[pl.ds(base0 + k * GC, GC)], so[b]).wait()


def _sc_gather(y1f, y2f, idx1, idx2):
    mesh = plsc.VectorSubcoreMesh(core_axis_name="c", subcore_axis_name="s")
    vm = lambda: pltpu.VMEM((GC, H), jnp.float32)
    vo = lambda: pltpu.VMEM((GC, H // 2), jnp.int32)
    f = pl.kernel(
        _sc_gather_body,
        out_type=jax.ShapeDtypeStruct((EH, H // 2), jnp.int32),
        mesh=mesh,
        compiler_params=pltpu.CompilerParams(needs_layout_passes=False),
        scratch_types=[
            pltpu.VMEM((EW,), jnp.int32),
            pltpu.VMEM((EW,), jnp.int32),
            [vm(), vm()],
            [vm(), vm()],
            [vo(), vo()],
            [pltpu.SemaphoreType.DMA, pltpu.SemaphoreType.DMA],
            [pltpu.SemaphoreType.DMA, pltpu.SemaphoreType.DMA],
            [pltpu.SemaphoreType.DMA, pltpu.SemaphoreType.DMA],
        ],
    )
    return f(y1f, y2f, idx1, idx2)


def _k3_body(h_ref, et_ref, w2_ref, b2_ref, out_ref):
    h = h_ref[...].astype(jnp.float32)
    m = jnp.dot(h, w2_ref[...], preferred_element_type=jnp.float32)
    et = et_ref[...]
    acc = jnp.zeros_like(out_ref)
    for t in range(T):
        mask = (et == t).astype(jnp.float32)
        acc = acc + mask * (m[:, t * H:(t + 1) * H] + b2_ref[0, t * H:(t + 1) * H])
    out_ref[...] = acc


def _second_layer(hmid, et2d, w2cat, b2flat):
    blk = 2000
    return pl.pallas_call(
        _k3_body,
        grid=(EH // blk,),
        in_specs=[
            pl.BlockSpec((blk, H), lambda i: (i, 0)),
            pl.BlockSpec((blk, 1), lambda i: (i, 0)),
            pl.BlockSpec((H, T * H), lambda i: (0, 0)),
            pl.BlockSpec((1, T * H), lambda i: (0, 0)),
        ],
        out_specs=pl.BlockSpec((blk, H), lambda i: (i, 0)),
        out_shape=jax.ShapeDtypeStruct((EH, H), jnp.float32),
    )(hmid, et2d, w2cat, b2flat)


def _sc_scatter_body(m0_hbm, dst_hbm, zrows_hbm, out_hbm,
                     dstb, buf, agg_sh, sd, sm):
    c = lax.axis_index("c")
    s = lax.axis_index("s")
    wid = s * NC + c

    pltpu.sync_copy(zrows_hbm, agg_sh.at[pl.ds(s * NROWS, NROWS)])

    @pl.when(s == 0)
    def _():
        pltpu.sync_copy(zrows_hbm.at[pl.ds(0, NREM)],
                        agg_sh.at[pl.ds(NS * NROWS, NREM)])

    plsc.subcore_barrier()

    for m_hbm in (m0_hbm,):
        mbase0 = wid * EW
        dbase0 = wid * EW

        def issue(k, b):
            pltpu.async_copy(dst_hbm.at[pl.ds(dbase0 + k * GC, GC)],
                             dstb[b], sd[b])
            pltpu.async_copy(m_hbm.at[pl.ds(mbase0 + k * GC, GC)],
                             buf[b], sm[b])

        issue(0, 0)
        issue(1, 1)

        def step(k, b):
            pltpu.make_async_copy(dst_hbm.at[pl.ds(dbase0 + k * GC, GC)],
                                  dstb[b], sd[b]).wait()
            pltpu.make_async_copy(m_hbm.at[pl.ds(mbase0 + k * GC, GC)],
                                  buf[b], sm[b]).wait()
            pltpu.sync_copy(buf[b], agg_sh.at[dstb[b]], add=True)

            @pl.when(k < NCHUNK - 2)
            def _():
                issue(k + 2, b)

        def chunk(k, carry):
            @pl.when(k % 2 == 0)
            def _():
                step(k, 0)

            @pl.when(k % 2 == 1)
            def _():
                step(k, 1)

            return carry

        lax.fori_loop(0, NCHUNK, chunk, 0)

    plsc.subcore_barrier()

    pltpu.sync_copy(agg_sh.at[pl.ds(s * NROWS, NROWS)],
                    out_hbm.at[c].at[pl.ds(s * NROWS, NROWS)])

    @pl.when(s == 0)
    def _():
        pltpu.sync_copy(agg_sh.at[pl.ds(NS * NROWS, NREM)],
                        out_hbm.at[c].at[pl.ds(NS * NROWS, NREM)])


def _sc_scatter(m0, dst, zrows):
    mesh = plsc.VectorSubcoreMesh(core_axis_name="c", subcore_axis_name="s")
    f = pl.kernel(
        _sc_scatter_body,
        out_type=jax.ShapeDtypeStruct((NC, N, H), jnp.float32),
        mesh=mesh,
        scratch_types=[
            [pltpu.VMEM((GC,), jnp.int32), pltpu.VMEM((GC,), jnp.int32)],
            [pltpu.VMEM((GC, H), jnp.float32), pltpu.VMEM((GC, H), jnp.float32)],
            pltpu.VMEM_SHARED((N, H), jnp.float32),
            [pltpu.SemaphoreType.DMA, pltpu.SemaphoreType.DMA],
            [pltpu.SemaphoreType.DMA, pltpu.SemaphoreType.DMA],
        ],
    )
    return f(m0, dst, zrows)


def _k5_body(p_ref, x_ref, wih_ref, bih_ref, whh_ref, bhh_ref, out_ref):
    agg = p_ref[0] + p_ref[1]
    x = x_ref[...]
    gi = jnp.dot(agg, wih_ref[...], preferred_element_type=jnp.float32) + bih_ref[...]
    gh = jnp.dot(x, whh_ref[...], preferred_element_type=jnp.float32) + bhh_ref[...]
    r = jax.nn.sigmoid(gi[:, :H] + gh[:, :H])
    z = jax.nn.sigmoid(gi[:, H:2 * H] + gh[:, H:2 * H])
    n = jnp.tanh(gi[:, 2 * H:] + r * gh[:, 2 * H:])
    out_ref[...] = (1.0 - z) * n + z * x


def _gru(partials, x, w_ih, b_ih, w_hh, b_hh):
    blk = 1000
    return pl.pallas_call(
        _k5_body,
        grid=(N // blk,),
        in_specs=[
            pl.BlockSpec((NC, blk, H), lambda i: (0, i, 0)),
            pl.BlockSpec((blk, H), lambda i: (i, 0)),
            pl.BlockSpec((H, 3 * H), lambda i: (0, 0)),
            pl.BlockSpec((1, 3 * H), lambda i: (0, 0)),
            pl.BlockSpec((H, 3 * H), lambda i: (0, 0)),
            pl.BlockSpec((1, 3 * H), lambda i: (0, 0)),
        ],
        out_specs=pl.BlockSpec((blk, H), lambda i: (i, 0)),
        out_shape=jax.ShapeDtypeStruct((N, H), jnp.float32),
    )(partials, x, w_ih, b_ih, w_hh, b_hh)


def kernel(x, edge_index, edge_type, W1, b1, W2, b2, W_ih, b_ih, W_hh, b_hh):
    src = edge_index[0]
    dst = edge_index[1]

    wsrc = jnp.transpose(W1[:, :H, :], (1, 0, 2)).reshape(H, T * H)
    wdst = jnp.transpose(W1[:, H:, :], (1, 0, 2)).reshape(H, T * H)
    b1flat = b1.reshape(1, T * H)
    w2cat = jnp.transpose(W2, (1, 0, 2)).reshape(H, T * H)
    b2flat = b2.reshape(1, T * H)

    y1, y2 = _precompute_y(x, wsrc, wdst, b1flat)
    y1f = y1.reshape(N * T, H)
    y2f = y2.reshape(N * T, H)

    rows = E // H
    idx1_2d, idx2_2d = _edge_indices(
        src.reshape(rows, H), dst.reshape(rows, H), edge_type.reshape(rows, H))

    idx1 = idx1_2d.reshape(E)
    idx2 = idx2_2d.reshape(E)
    et2d = edge_type.reshape(E, 1)

    h_i32 = _sc_gather(y1f, y2f, idx1, idx2)
    h16 = jax.lax.bitcast_convert_type(h_i32, jnp.bfloat16).reshape(E, H)
    w2p = w2cat[jnp.array(_PERM), :]
    msgs = _second_layer(h16, et2d, w2p, b2flat)

    zrows = jnp.zeros((NROWS, H), jnp.float32)
    partials = _sc_scatter(msgs, dst, zrows)

    return _gru(partials, x, W_ih, b_ih.reshape(1, 3 * H), W_hh,
                b_hh.reshape(1, 3 * H))


# speedup vs baseline: 1.9745x; 1.9745x over previous
import functools

import jax
import jax.numpy as jnp
from jax import lax
from jax.experimental import pallas as pl
from jax.experimental.pallas import tpu as pltpu
from jax.experimental.pallas import tpu_sc as plsc

N = 10000
E = 320000
H = 128
T = 4

NC = 2
NS = 16
NW = NC * NS
EH = E
EW = EH // NW
GC = 80
NCHUNK = EW // GC
NROWS = 624
NREM = N - NS * NROWS

_PERM = []
for _g in range(4):
    for _i in range(16):
        _PERM.append(_g * 32 + _i)
        _PERM.append(_g * 32 + 16 + _i)


def _k1_body(x_ref, wsrc_ref, wdst_ref, b1_ref, y1_ref, y2_ref):
    x = x_ref[...]
    y1_ref[...] = jnp.dot(x, wsrc_ref[...], preferred_element_type=jnp.float32)
    y2_ref[...] = (
        jnp.dot(x, wdst_ref[...], preferred_element_type=jnp.float32)
        + b1_ref[...]
    )


def _precompute_y(x, wsrc, wdst, b1flat):
    blk = 1000
    return pl.pallas_call(
        _k1_body,
        grid=(N // blk,),
        in_specs=[
            pl.BlockSpec((blk, H), lambda i: (i, 0)),
            pl.BlockSpec((H, T * H), lambda i: (0, 0)),
            pl.BlockSpec((H, T * H), lambda i: (0, 0)),
            pl.BlockSpec((1, T * H), lambda i: (0, 0)),
        ],
        out_specs=[
            pl.BlockSpec((blk, T * H), lambda i: (i, 0)),
            pl.BlockSpec((blk, T * H), lambda i: (i, 0)),
        ],
        out_shape=[
            jax.ShapeDtypeStruct((N, T * H), jnp.float32),
            jax.ShapeDtypeStruct((N, T * H), jnp.float32),
        ],
    )(x, wsrc, wdst, b1flat)


def _k1b_body(src_ref, dst_ref, et_ref, i1_ref, i2_ref):
    et = et_ref[...]
    i1_ref[...] = src_ref[...] * T + et
    i2_ref[...] = dst_ref[...] * T + et


def _edge_indices(src2d, dst2d, et2d):
    rows = E // H
    return pl.pallas_call(
        _k1b_body,
        grid=(1,),
        in_specs=[pl.BlockSpec((rows, H), lambda i: (0, 0))] * 3,
        out_specs=[pl.BlockSpec((rows, H), lambda i: (0, 0))] * 2,
        out_shape=[jax.ShapeDtypeStruct((rows, H), jnp.int32)] * 2,
    )(src2d, dst2d, et2d)


def _sc_gather_body(y1_hbm, y2_hbm, i1_hbm, i2_hbm, hm_hbm,
                    i1_all, i2_all, abuf, bbuf, obuf, sa, sb, so):
    c = lax.axis_index("c")
    s = lax.axis_index("s")
    wid = s * NC + c
    gbase = wid * EW
    base0 = wid * EW

    pltpu.sync_copy(i1_hbm.at[pl.ds(gbase, EW)], i1_all)
    pltpu.sync_copy(i2_hbm.at[pl.ds(gbase, EW)], i2_all)

    def issue(k, b):
        off = k * GC
        pltpu.async_copy(y1_hbm.at[i1_all.at[pl.ds(off, GC)]], abuf[b], sa[b])
        pltpu.async_copy(y2_hbm.at[i2_all.at[pl.ds(off, GC)]], bbuf[b], sb[b])

    issue(0, 0)
    issue(1, 1)

    def step(k, b):
        off = k * GC
        pltpu.make_async_copy(y1_hbm.at[i1_all.at[pl.ds(off, GC)]],
                              abuf[b], sa[b]).wait()
        pltpu.make_async_copy(y2_hbm.at[i2_all.at[pl.ds(off, GC)]],
                              bbuf[b], sb[b]).wait()

        @pl.when(k >= 2)
        def _():
            pltpu.make_async_copy(
                obuf[b], hm_hbm.at[pl.ds(base0 + (k - 2) * GC, GC)],
                so[b]).wait()

        def rows(r, carry):
            for g in range(H // 32):
                sl0 = pl.ds(g * 32, 16)
                sl1 = pl.ds(g * 32 + 16, 16)
                s0 = jnp.maximum(abuf[b][r, sl0] + bbuf[b][r, sl0], 0.0)
                s1 = jnp.maximum(abuf[b][r, sl1] + bbuf[b][r, sl1], 0.0)
                p = plsc.pack(s0, s1, format=plsc.PackFormat.INTERLEAVED)
                obuf[b][r, pl.ds(g * 16, 16)] = plsc.bitcast(p, jnp.int32)
            return carry

        lax.fori_loop(0, GC, rows, 0)

        @pl.when(k < NCHUNK - 2)
        def _():
            issue(k + 2, b)

        pltpu.async_copy(obuf[b], hm_hbm.at[pl.ds(base0 + off, GC)], so[b])

    def chunk(k, carry):
        @pl.when(k % 2 == 0)
        def _():
            step(k, 0)

        @pl.when(k % 2 == 1)
        def _():
            step(k, 1)

        return carry

    lax.fori_loop(0, NCHUNK, chunk, 0)

    for b, k in ((0, NCHUNK - 1), (1, NCHUNK - 2)):
        pltpu.make_async_copy(
            obuf[b], hm_hbm.at[pl.ds(base0 + k * GC, GC)], so[b]).wait()


def _sc_gather(y1f, y2f, idx1, idx2):
    mesh = plsc.VectorSubcoreMesh(core_axis_name="c", subcore_axis_name="s")
    vm = lambda: pltpu.VMEM((GC, H), jnp.float32)
    vo = lambda: pltpu.VMEM((GC, H // 2), jnp.int32)
    f = pl.kernel(
        _sc_gather_body,
        out_type=jax.ShapeDtypeStruct((EH, H // 2), jnp.int32),
        mesh=mesh,
        compiler_params=pltpu.CompilerParams(needs_layout_passes=False),
        scratch_types=[
            pltpu.VMEM((EW,), jnp.int32),
            pltpu.VMEM((EW,), jnp.int32),
            [vm(), vm()],
            [vm(), vm()],
            [vo(), vo()],
            [pltpu.SemaphoreType.DMA, pltpu.SemaphoreType.DMA],
            [pltpu.SemaphoreType.DMA, pltpu.SemaphoreType.DMA],
            [pltpu.SemaphoreType.DMA, pltpu.SemaphoreType.DMA],
        ],
    )
    return f(y1f, y2f, idx1, idx2)


def _k3_body(h_ref, et_ref, w2e_ref, w2o_ref, b2_ref, out_ref):
    w = h_ref[...]
    he = jax.lax.bitcast_convert_type(jnp.left_shift(w, 16), jnp.float32)
    ho = jax.lax.bitcast_convert_type(
        jnp.bitwise_and(w, jnp.int32(-65536)), jnp.float32)
    m = (jnp.dot(he, w2e_ref[...], preferred_element_type=jnp.float32)
         + jnp.dot(ho, w2o_ref[...], preferred_element_type=jnp.float32))
    et = et_ref[...]
    acc = jnp.zeros_like(out_ref)
    for t in range(T):
        mask = (et == t).astype(jnp.float32)
        acc = acc + mask * (m[:, t * H:(t + 1) * H] + b2_ref[0, t * H:(t + 1) * H])
    out_ref[...] = acc


def _second_layer(h_i32, et2d, w2e, w2o, b2flat):
    blk = 2000
    return pl.pallas_call(
        _k3_body,
        grid=(EH // blk,),
        in_specs=[
            pl.BlockSpec((blk, H // 2), lambda i: (i, 0)),
            pl.BlockSpec((blk, 1), lambda i: (i, 0)),
            pl.BlockSpec((H // 2, T * H), lambda i: (0, 0)),
            pl.BlockSpec((H // 2, T * H), lambda i: (0, 0)),
            pl.BlockSpec((1, T * H), lambda i: (0, 0)),
        ],
        out_specs=pl.BlockSpec((blk, H), lambda i: (i, 0)),
        out_shape=jax.ShapeDtypeStruct((EH, H), jnp.float32),
    )(h_i32, et2d, w2e, w2o, b2flat)


def _sc_scatter_body(m0_hbm, dst_hbm, zrows_hbm, out_hbm,
                     dstb, buf, agg_sh, sd, sm):
    c = lax.axis_index("c")
    s = lax.axis_index("s")
    wid = s * NC + c

    pltpu.sync_copy(zrows_hbm, agg_sh.at[pl.ds(s * NROWS, NROWS)])

    @pl.when(s == 0)
    def _():
        pltpu.sync_copy(zrows_hbm.at[pl.ds(0, NREM)],
                        agg_sh.at[pl.ds(NS * NROWS, NREM)])

    plsc.subcore_barrier()

    for m_hbm in (m0_hbm,):
        mbase0 = wid * EW
        dbase0 = wid * EW

        def issue(k, b):
            pltpu.async_copy(dst_hbm.at[pl.ds(dbase0 + k * GC, GC)],
                             dstb[b], sd[b])
            pltpu.async_copy(m_hbm.at[pl.ds(mbase0 + k * GC, GC)],
                             buf[b], sm[b])

        issue(0, 0)
        issue(1, 1)

        def step(k, b):
            pltpu.make_async_copy(dst_hbm.at[pl.ds(dbase0 + k * GC, GC)],
                                  dstb[b], sd[b]).wait()
            pltpu.make_async_copy(m_hbm.at[pl.ds(mbase0 + k * GC, GC)],
                                  buf[b], sm[b]).wait()
            pltpu.sync_copy(buf[b], agg_sh.at[dstb[b]], add=True)

            @pl.when(k < NCHUNK - 2)
            def _():
                issue(k + 2, b)

        def chunk(k, carry):
            @pl.when(k % 2 == 0)
            def _():
                step(k, 0)

            @pl.when(k % 2 == 1)
            def _():
                step(k, 1)

            return carry

        lax.fori_loop(0, NCHUNK, chunk, 0)

    plsc.subcore_barrier()

    pltpu.sync_copy(agg_sh.at[pl.ds(s * NROWS, NROWS)],
                    out_hbm.at[c].at[pl.ds(s * NROWS, NROWS)])

    @pl.when(s == 0)
    def _():
        pltpu.sync_copy(agg_sh.at[pl.ds(NS * NROWS, NREM)],
                        out_hbm.at[c].at[pl.ds(NS * NROWS, NREM)])


def _sc_scatter(m0, dst, zrows):
    mesh = plsc.VectorSubcoreMesh(core_axis_name="c", subcore_axis_name="s")
    f = pl.kernel(
        _sc_scatter_body,
        out_type=jax.ShapeDtypeStruct((NC, N, H), jnp.float32),
        mesh=mesh,
        scratch_types=[
            [pltpu.VMEM((GC,), jnp.int32), pltpu.VMEM((GC,), jnp.int32)],
            [pltpu.VMEM((GC, H), jnp.float32), pltpu.VMEM((GC, H), jnp.float32)],
            pltpu.VMEM_SHARED((N, H), jnp.float32),
            [pltpu.SemaphoreType.DMA, pltpu.SemaphoreType.DMA],
            [pltpu.SemaphoreType.DMA, pltpu.SemaphoreType.DMA],
        ],
    )
    return f(m0, dst, zrows)


def _k5_body(p_ref, x_ref, wih_ref, bih_ref, whh_ref, bhh_ref, out_ref):
    agg = p_ref[0] + p_ref[1]
    x = x_ref[...]
    gi = jnp.dot(agg, wih_ref[...], preferred_element_type=jnp.float32) + bih_ref[...]
    gh = jnp.dot(x, whh_ref[...], preferred_element_type=jnp.float32) + bhh_ref[...]
    r = jax.nn.sigmoid(gi[:, :H] + gh[:, :H])
    z = jax.nn.sigmoid(gi[:, H:2 * H] + gh[:, H:2 * H])
    n = jnp.tanh(gi[:, 2 * H:] + r * gh[:, 2 * H:])
    out_ref[...] = (1.0 - z) * n + z * x


def _gru(partials, x, w_ih, b_ih, w_hh, b_hh):
    blk = 1000
    return pl.pallas_call(
        _k5_body,
        grid=(N // blk,),
        in_specs=[
            pl.BlockSpec((NC, blk, H), lambda i: (0, i, 0)),
            pl.BlockSpec((blk, H), lambda i: (i, 0)),
            pl.BlockSpec((H, 3 * H), lambda i: (0, 0)),
            pl.BlockSpec((1, 3 * H), lambda i: (0, 0)),
            pl.BlockSpec((H, 3 * H), lambda i: (0, 0)),
            pl.BlockSpec((1, 3 * H), lambda i: (0, 0)),
        ],
        out_specs=pl.BlockSpec((blk, H), lambda i: (i, 0)),
        out_shape=jax.ShapeDtypeStruct((N, H), jnp.float32),
    )(partials, x, w_ih, b_ih, w_hh, b_hh)


def kernel(x, edge_index, edge_type, W1, b1, W2, b2, W_ih, b_ih, W_hh, b_hh):
    src = edge_index[0]
    dst = edge_index[1]

    wsrc = jnp.transpose(W1[:, :H, :], (1, 0, 2)).reshape(H, T * H)
    wdst = jnp.transpose(W1[:, H:, :], (1, 0, 2)).reshape(H, T * H)
    b1flat = b1.reshape(1, T * H)
    w2cat = jnp.transpose(W2, (1, 0, 2)).reshape(H, T * H)
    b2flat = b2.reshape(1, T * H)

    y1, y2 = _precompute_y(x, wsrc, wdst, b1flat)
    y1f = y1.reshape(N * T, H)
    y2f = y2.reshape(N * T, H)

    rows = E // H
    idx1_2d, idx2_2d = _edge_indices(
        src.reshape(rows, H), dst.reshape(rows, H), edge_type.reshape(rows, H))

    idx1 = idx1_2d.reshape(E)
    idx2 = idx2_2d.reshape(E)
    et2d = edge_type.reshape(E, 1)

    h_i32 = _sc_gather(y1f, y2f, idx1, idx2)
    w2e = w2cat[jnp.array(_PERM[0::2]), :]
    w2o = w2cat[jnp.array(_PERM[1::2]), :]
    msgs = _second_layer(h_i32, et2d, w2e, w2o, b2flat)

    zrows = jnp.zeros((NROWS, H), jnp.float32)
    partials = _sc_scatter(msgs, dst, zrows)

    return _gru(partials, x, W_ih, b_ih.reshape(1, 3 * H), W_hh,
                b_hh.reshape(1, 3 * H))


# speedup vs baseline: 1.9930x; 1.0094x over previous
import functools

import jax
import jax.numpy as jnp
from jax import lax
from jax.experimental import pallas as pl
from jax.experimental.pallas import tpu as pltpu
from jax.experimental.pallas import tpu_sc as plsc

N = 10000
E = 320000
H = 128
T = 4

NC = 2
NS = 16
NW = NC * NS
EH = E
EW = EH // NW
GC = 80
NCHUNK = EW // GC
NROWS = 624
NREM = N - NS * NROWS

_PERM = []
for _g in range(4):
    for _i in range(16):
        _PERM.append(_g * 32 + _i)
        _PERM.append(_g * 32 + 16 + _i)


def _k1_body(x_ref, wsrc_ref, wdst_ref, b1_ref, y1_ref, y2_ref):
    x = x_ref[...]
    y1 = jnp.dot(x, wsrc_ref[...], preferred_element_type=jnp.float32)
    y2 = (jnp.dot(x, wdst_ref[...], preferred_element_type=jnp.float32)
          + b1_ref[...])
    y1_ref[...] = y1
    y2_ref[...] = y2


def _precompute_y(x, wsrc, wdst, b1flat):
    blk = 1000
    return pl.pallas_call(
        _k1_body,
        grid=(N // blk,),
        in_specs=[
            pl.BlockSpec((blk, H), lambda i: (i, 0)),
            pl.BlockSpec((H, T * H), lambda i: (0, 0)),
            pl.BlockSpec((H, T * H), lambda i: (0, 0)),
            pl.BlockSpec((1, T * H), lambda i: (0, 0)),
        ],
        out_specs=[
            pl.BlockSpec((blk, T * H), lambda i: (i, 0)),
            pl.BlockSpec((blk, T * H), lambda i: (i, 0)),
        ],
        out_shape=[
            jax.ShapeDtypeStruct((N, T * H), jnp.float32),
            jax.ShapeDtypeStruct((N, T * H), jnp.float32),
        ],
    )(x, wsrc, wdst, b1flat)


def _k1b_body(src_ref, dst_ref, et_ref, i1_ref, i2_ref):
    et = et_ref[...]
    i1_ref[...] = src_ref[...] * T + et
    i2_ref[...] = dst_ref[...] * T + et


def _edge_indices(src2d, dst2d, et2d):
    rows = E // H
    return pl.pallas_call(
        _k1b_body,
        grid=(1,),
        in_specs=[pl.BlockSpec((rows, H), lambda i: (0, 0))] * 3,
        out_specs=[pl.BlockSpec((rows, H), lambda i: (0, 0))] * 2,
        out_shape=[jax.ShapeDtypeStruct((rows, H), jnp.int32)] * 2,
    )(src2d, dst2d, et2d)


def _sc_gather_body(y1_hbm, y2_hbm, i1_hbm, i2_hbm, hm_hbm,
                    i1_all, i2_all, abuf, bbuf, obuf, sa, sb, so):
    c = lax.axis_index("c")
    s = lax.axis_index("s")
    wid = s * NC + c
    gbase = wid * EW
    base0 = wid * EW

    pltpu.sync_copy(i1_hbm.at[pl.ds(gbase, EW)], i1_all)
    pltpu.sync_copy(i2_hbm.at[pl.ds(gbase, EW)], i2_all)

    def issue(k, b):
        off = k * GC
        pltpu.async_copy(y1_hbm.at[i1_all.at[pl.ds(off, GC)]], abuf[b], sa[b])
        pltpu.async_copy(y2_hbm.at[i2_all.at[pl.ds(off, GC)]], bbuf[b], sb[b])

    issue(0, 0)
    issue(1, 1)

    def step(k, b):
        off = k * GC
        pltpu.make_async_copy(y1_hbm.at[i1_all.at[pl.ds(off, GC)]],
                              abuf[b], sa[b]).wait()
        pltpu.make_async_copy(y2_hbm.at[i2_all.at[pl.ds(off, GC)]],
                              bbuf[b], sb[b]).wait()

        @pl.when(k >= 2)
        def _():
            pltpu.make_async_copy(
                obuf[b], hm_hbm.at[pl.ds(base0 + (k - 2) * GC, GC)],
                so[b]).wait()

        def rows(r, carry):
            for g in range(H // 32):
                sl0 = pl.ds(g * 32, 16)
                sl1 = pl.ds(g * 32 + 16, 16)
                s0 = jnp.maximum(abuf[b][r, sl0] + bbuf[b][r, sl0], 0.0)
                s1 = jnp.maximum(abuf[b][r, sl1] + bbuf[b][r, sl1], 0.0)
                p = plsc.pack(s0, s1, format=plsc.PackFormat.INTERLEAVED)
                obuf[b][r, pl.ds(g * 16, 16)] = plsc.bitcast(p, jnp.int32)
            return carry

        lax.fori_loop(0, GC, rows, 0)

        @pl.when(k < NCHUNK - 2)
        def _():
            issue(k + 2, b)

        pltpu.async_copy(obuf[b], hm_hbm.at[pl.ds(base0 + off, GC)], so[b])

    def chunk(k, carry):
        @pl.when(k % 2 == 0)
        def _():
            step(k, 0)

        @pl.when(k % 2 == 1)
        def _():
            step(k, 1)

        return carry

    lax.fori_loop(0, NCHUNK, chunk, 0)

    for b, k in ((0, NCHUNK - 1), (1, NCHUNK - 2)):
        pltpu.make_async_copy(
            obuf[b], hm_hbm.at[pl.ds(base0 + k * GC, GC)], so[b]).wait()


def _sc_gather(y1f, y2f, idx1, idx2):
    mesh = plsc.VectorSubcoreMesh(core_axis_name="c", subcore_axis_name="s")
    vm = lambda: pltpu.VMEM((GC, H), jnp.float32)
    vo = lambda: pltpu.VMEM((GC, H // 2), jnp.int32)
    f = pl.kernel(
        _sc_gather_body,
        out_type=jax.ShapeDtypeStruct((EH, H // 2), jnp.int32),
        mesh=mesh,
        compiler_params=pltpu.CompilerParams(needs_layout_passes=False),
        scratch_types=[
            pltpu.VMEM((EW,), jnp.int32),
            pltpu.VMEM((EW,), jnp.int32),
            [vm(), vm()],
            [vm(), vm()],
            [vo(), vo()],
            [pltpu.SemaphoreType.DMA, pltpu.SemaphoreType.DMA],
            [pltpu.SemaphoreType.DMA, pltpu.SemaphoreType.DMA],
            [pltpu.SemaphoreType.DMA, pltpu.SemaphoreType.DMA],
        ],
    )
    return f(y1f, y2f, idx1, idx2)


def _k3_body(h_ref, et_ref, w2_ref, b2_ref, out_ref):
    w = h_ref[...]
    hlo = jax.lax.bitcast_convert_type(jnp.left_shift(w, 16), jnp.float32)
    hhi = jax.lax.bitcast_convert_type(
        jnp.bitwise_and(w, jnp.int32(-65536)), jnp.float32)
    h = jnp.concatenate([hlo, hhi], axis=1)
    m = jnp.dot(h, w2_ref[...], preferred_element_type=jnp.float32)
    et = et_ref[...]
    acc = jnp.zeros_like(out_ref)
    for t in range(T):
        mask = (et == t).astype(jnp.float32)
        acc = acc + mask * (m[:, t * H:(t + 1) * H] + b2_ref[0, t * H:(t + 1) * H])
    out_ref[...] = acc


def _second_layer(h_i32, et2d, w2cat, b2flat):
    blk = 2000
    return pl.pallas_call(
        _k3_body,
        grid=(EH // blk,),
        in_specs=[
            pl.BlockSpec((blk, H // 2), lambda i: (i, 0)),
            pl.BlockSpec((blk, 1), lambda i: (i, 0)),
            pl.BlockSpec((H, T * H), lambda i: (0, 0)),
            pl.BlockSpec((1, T * H), lambda i: (0, 0)),
        ],
        out_specs=pl.BlockSpec((blk, H), lambda i: (i, 0)),
        out_shape=jax.ShapeDtypeStruct((EH, H), jnp.float32),
    )(h_i32, et2d, w2cat, b2flat)


def _sc_scatter_body(m0_hbm, dst_hbm, zrows_hbm, out_hbm,
                     dstb, buf, agg_sh, sd, sm):
    c = lax.axis_index("c")
    s = lax.axis_index("s")
    wid = s * NC + c

    pltpu.sync_copy(zrows_hbm, agg_sh.at[pl.ds(s * NROWS, NROWS)])

    @pl.when(s == 0)
    def _():
        pltpu.sync_copy(zrows_hbm.at[pl.ds(0, NREM)],
                        agg_sh.at[pl.ds(NS * NROWS, NREM)])

    plsc.subcore_barrier()

    for m_hbm in (m0_hbm,):
        mbase0 = wid * EW
        dbase0 = wid * EW

        def issue(k, b):
            pltpu.async_copy(dst_hbm.at[pl.ds(dbase0 + k * GC, GC)],
                             dstb[b], sd[b])
            pltpu.async_copy(m_hbm.at[pl.ds(mbase0 + k * GC, GC)],
                             buf[b], sm[b])

        issue(0, 0)
        issue(1, 1)

        def step(k, b):
            pltpu.make_async_copy(dst_hbm.at[pl.ds(dbase0 + k * GC, GC)],
                                  dstb[b], sd[b]).wait()
            pltpu.make_async_copy(m_hbm.at[pl.ds(mbase0 + k * GC, GC)],
                                  buf[b], sm[b]).wait()
            pltpu.sync_copy(buf[b], agg_sh.at[dstb[b]], add=True)

            @pl.when(k < NCHUNK - 2)
            def _():
                issue(k + 2, b)

        def chunk(k, carry):
            @pl.when(k % 2 == 0)
            def _():
                step(k, 0)

            @pl.when(k % 2 == 1)
            def _():
                step(k, 1)

            return carry

        lax.fori_loop(0, NCHUNK, chunk, 0)

    plsc.subcore_barrier()

    pltpu.sync_copy(agg_sh.at[pl.ds(s * NROWS, NROWS)],
                    out_hbm.at[c].at[pl.ds(s * NROWS, NROWS)])

    @pl.when(s == 0)
    def _():
        pltpu.sync_copy(agg_sh.at[pl.ds(NS * NROWS, NREM)],
                        out_hbm.at[c].at[pl.ds(NS * NROWS, NREM)])


def _sc_scatter(m0, dst, zrows):
    mesh = plsc.VectorSubcoreMesh(core_axis_name="c", subcore_axis_name="s")
    f = pl.kernel(
        _sc_scatter_body,
        out_type=jax.ShapeDtypeStruct((NC, N, H), jnp.float32),
        mesh=mesh,
        scratch_types=[
            [pltpu.VMEM((GC,), jnp.int32), pltpu.VMEM((GC,), jnp.int32)],
            [pltpu.VMEM((GC, H), jnp.float32), pltpu.VMEM((GC, H), jnp.float32)],
            pltpu.VMEM_SHARED((N, H), jnp.float32),
            [pltpu.SemaphoreType.DMA, pltpu.SemaphoreType.DMA],
            [pltpu.SemaphoreType.DMA, pltpu.SemaphoreType.DMA],
        ],
    )
    return f(m0, dst, zrows)


def _k5_body(p_ref, x_ref, wih_ref, bih_ref, whh_ref, bhh_ref, out_ref):
    agg = p_ref[0] + p_ref[1]
    x = x_ref[...]
    gi = jnp.dot(agg, wih_ref[...], preferred_element_type=jnp.float32) + bih_ref[...]
    gh = jnp.dot(x, whh_ref[...], preferred_element_type=jnp.float32) + bhh_ref[...]
    r = jax.nn.sigmoid(gi[:, :H] + gh[:, :H])
    z = jax.nn.sigmoid(gi[:, H:2 * H] + gh[:, H:2 * H])
    n = jnp.tanh(gi[:, 2 * H:] + r * gh[:, 2 * H:])
    out_ref[...] = (1.0 - z) * n + z * x


def _gru(partials, x, w_ih, b_ih, w_hh, b_hh):
    blk = 1000
    return pl.pallas_call(
        _k5_body,
        grid=(N // blk,),
        in_specs=[
            pl.BlockSpec((NC, blk, H), lambda i: (0, i, 0)),
            pl.BlockSpec((blk, H), lambda i: (i, 0)),
            pl.BlockSpec((H, 3 * H), lambda i: (0, 0)),
            pl.BlockSpec((1, 3 * H), lambda i: (0, 0)),
            pl.BlockSpec((H, 3 * H), lambda i: (0, 0)),
            pl.BlockSpec((1, 3 * H), lambda i: (0, 0)),
        ],
        out_specs=pl.BlockSpec((blk, H), lambda i: (i, 0)),
        out_shape=jax.ShapeDtypeStruct((N, H), jnp.float32),
    )(partials, x, w_ih, b_ih, w_hh, b_hh)


def kernel(x, edge_index, edge_type, W1, b1, W2, b2, W_ih, b_ih, W_hh, b_hh):
    src = edge_index[0]
    dst = edge_index[1]

    wsrc = jnp.transpose(W1[:, :H, :], (1, 0, 2)).reshape(H, T * H)
    wdst = jnp.transpose(W1[:, H:, :], (1, 0, 2)).reshape(H, T * H)
    b1flat = b1.reshape(1, T * H)
    w2cat = jnp.transpose(W2, (1, 0, 2)).reshape(H, T * H)
    b2flat = b2.reshape(1, T * H)

    y1, y2 = _precompute_y(x, wsrc, wdst, b1flat)
    y1f = y1.reshape(N * T, H)
    y2f = y2.reshape(N * T, H)

    rows = E // H
    idx1_2d, idx2_2d = _edge_indices(
        src.reshape(rows, H), dst.reshape(rows, H), edge_type.reshape(rows, H))

    idx1 = idx1_2d.reshape(E)
    idx2 = idx2_2d.reshape(E)
    et2d = edge_type.reshape(E, 1)

    h_i32 = _sc_gather(y1f, y2f, idx1, idx2)
    w2p = w2cat[jnp.array(_PERM[0::2] + _PERM[1::2]), :]
    msgs = _second_layer(h_i32, et2d, w2p, b2flat)

    zrows = jnp.zeros((NROWS, H), jnp.float32)
    partials = _sc_scatter(msgs, dst, zrows)

    return _gru(partials, x, W_ih, b_ih.reshape(1, 3 * H), W_hh,
                b_hh.reshape(1, 3 * H))


# speedup vs baseline: 2.0543x; 1.0307x over previous
import functools

import jax
import jax.numpy as jnp
from jax import lax
from jax.experimental import pallas as pl
from jax.experimental.pallas import tpu as pltpu
from jax.experimental.pallas import tpu_sc as plsc

N = 10000
E = 320000
H = 128
T = 4

NC = 2
NS = 16
NW = NC * NS
EH = E
EW = EH // NW
GC = 80
NCHUNK = EW // GC
NROWS = 624
NREM = N - NS * NROWS

_PERM = []
for _g in range(4):
    for _i in range(16):
        _PERM.append(_g * 32 + _i)
        _PERM.append(_g * 32 + 16 + _i)


def _k1_body(x_ref, wsrc_ref, wdst_ref, b1_ref, y1_ref, y2_ref):
    x = x_ref[...]
    y1 = jnp.dot(x, wsrc_ref[...], preferred_element_type=jnp.float32)
    y2 = (jnp.dot(x, wdst_ref[...], preferred_element_type=jnp.float32)
          + b1_ref[...])
    y1_ref[...] = y1
    y2_ref[...] = y2


def _precompute_y(x, wsrc, wdst, b1flat):
    blk = 1000
    return pl.pallas_call(
        _k1_body,
        grid=(N // blk,),
        in_specs=[
            pl.BlockSpec((blk, H), lambda i: (i, 0)),
            pl.BlockSpec((H, T * H), lambda i: (0, 0)),
            pl.BlockSpec((H, T * H), lambda i: (0, 0)),
            pl.BlockSpec((1, T * H), lambda i: (0, 0)),
        ],
        out_specs=[
            pl.BlockSpec((blk, T * H), lambda i: (i, 0)),
            pl.BlockSpec((blk, T * H), lambda i: (i, 0)),
        ],
        out_shape=[
            jax.ShapeDtypeStruct((N, T * H), jnp.float32),
            jax.ShapeDtypeStruct((N, T * H), jnp.float32),
        ],
    )(x, wsrc, wdst, b1flat)


def _k1b_body(src_ref, dst_ref, et_ref, i1_ref, i2_ref):
    et = et_ref[...]
    i1_ref[...] = src_ref[...] * T + et
    i2_ref[...] = dst_ref[...] * T + et


def _edge_indices(src2d, dst2d, et2d):
    rows = E // H
    return pl.pallas_call(
        _k1b_body,
        grid=(1,),
        in_specs=[pl.BlockSpec((rows, H), lambda i: (0, 0))] * 3,
        out_specs=[pl.BlockSpec((rows, H), lambda i: (0, 0))] * 2,
        out_shape=[jax.ShapeDtypeStruct((rows, H), jnp.int32)] * 2,
    )(src2d, dst2d, et2d)


def _sc_gather_body(y1_hbm, y2_hbm, i1_hbm, i2_hbm, hm_hbm,
                    i1_all, i2_all, abuf, bbuf, obuf, sa, sb, so):
    c = lax.axis_index("c")
    s = lax.axis_index("s")
    wid = s * NC + c
    gbase = wid * EW
    base0 = wid * EW

    pltpu.sync_copy(i1_hbm.at[pl.ds(gbase, EW)], i1_all)
    pltpu.sync_copy(i2_hbm.at[pl.ds(gbase, EW)], i2_all)

    def issue(k, b):
        off = k * GC
        pltpu.async_copy(y1_hbm.at[i1_all.at[pl.ds(off, GC)]], abuf[b], sa[b])
        pltpu.async_copy(y2_hbm.at[i2_all.at[pl.ds(off, GC)]], bbuf[b], sb[b])

    issue(0, 0)
    issue(1, 1)

    def step(k, b):
        off = k * GC
        pltpu.make_async_copy(y1_hbm.at[i1_all.at[pl.ds(off, GC)]],
                              abuf[b], sa[b]).wait()
        pltpu.make_async_copy(y2_hbm.at[i2_all.at[pl.ds(off, GC)]],
                              bbuf[b], sb[b]).wait()

        @pl.when(k >= 2)
        def _():
            pltpu.make_async_copy(
                obuf[b], hm_hbm.at[pl.ds(base0 + (k - 2) * GC, GC)],
                so[b]).wait()

        def rows(r, carry):
            for g in range(H // 32):
                sl0 = pl.ds(g * 32, 16)
                sl1 = pl.ds(g * 32 + 16, 16)
                s0 = jnp.maximum(abuf[b][r, sl0] + bbuf[b][r, sl0], 0.0)
                s1 = jnp.maximum(abuf[b][r, sl1] + bbuf[b][r, sl1], 0.0)
                p = plsc.pack(s0, s1, format=plsc.PackFormat.INTERLEAVED)
                obuf[b][r, pl.ds(g * 16, 16)] = plsc.bitcast(p, jnp.int32)
            return carry

        lax.fori_loop(0, GC, rows, 0)

        @pl.when(k < NCHUNK - 2)
        def _():
            issue(k + 2, b)

        pltpu.async_copy(obuf[b], hm_hbm.at[pl.ds(base0 + off, GC)], so[b])

    def chunk(k, carry):
        @pl.when(k % 2 == 0)
        def _():
            step(k, 0)

        @pl.when(k % 2 == 1)
        def _():
            step(k, 1)

        return carry

    lax.fori_loop(0, NCHUNK, chunk, 0)

    for b, k in ((0, NCHUNK - 1), (1, NCHUNK - 2)):
        pltpu.make_async_copy(
            obuf[b], hm_hbm.at[pl.ds(base0 + k * GC, GC)], so[b]).wait()


def _sc_gather(y1f, y2f, idx1, idx2):
    mesh = plsc.VectorSubcoreMesh(core_axis_name="c", subcore_axis_name="s")
    vm = lambda: pltpu.VMEM((GC, H), jnp.float32)
    vo = lambda: pltpu.VMEM((GC, H // 2), jnp.int32)
    f = pl.kernel(
        _sc_gather_body,
        out_type=jax.ShapeDtypeStruct((EH, H // 2), jnp.int32),
        mesh=mesh,
        compiler_params=pltpu.CompilerParams(needs_layout_passes=False),
        scratch_types=[
            pltpu.VMEM((EW,), jnp.int32),
            pltpu.VMEM((EW,), jnp.int32),
            [vm(), vm()],
            [vm(), vm()],
            [vo(), vo()],
            [pltpu.SemaphoreType.DMA, pltpu.SemaphoreType.DMA],
            [pltpu.SemaphoreType.DMA, pltpu.SemaphoreType.DMA],
            [pltpu.SemaphoreType.DMA, pltpu.SemaphoreType.DMA],
        ],
    )
    return f(y1f, y2f, idx1, idx2)


def _k3_body(h_ref, et_ref, w2_ref, b2_ref, out_ref):
    w = h_ref[...]
    hlo = jax.lax.bitcast_convert_type(jnp.left_shift(w, 16), jnp.float32)
    hhi = jax.lax.bitcast_convert_type(
        jnp.bitwise_and(w, jnp.int32(-65536)), jnp.float32)
    h = jnp.concatenate([hlo, hhi], axis=1)
    m = jnp.dot(h, w2_ref[...], preferred_element_type=jnp.float32)
    et = et_ref[...]
    b0 = jnp.bitwise_and(et, 1) == 1
    b1 = jnp.bitwise_and(et, 2) == 2
    m01 = jnp.where(b0, m[:, H:2 * H], m[:, :H])
    m23 = jnp.where(b0, m[:, 3 * H:], m[:, 2 * H:3 * H])
    sel = jnp.where(b1, m23, m01)
    b2 = b2_ref[...]
    b01 = jnp.where(b0, b2[0:1, H:2 * H], b2[0:1, :H])
    b23 = jnp.where(b0, b2[0:1, 3 * H:], b2[0:1, 2 * H:3 * H])
    out_ref[...] = sel + jnp.where(b1, b23, b01)


def _second_layer(h_i32, et2d, w2cat, b2flat):
    blk = 2000
    return pl.pallas_call(
        _k3_body,
        grid=(EH // blk,),
        in_specs=[
            pl.BlockSpec((blk, H // 2), lambda i: (i, 0)),
            pl.BlockSpec((blk, 1), lambda i: (i, 0)),
            pl.BlockSpec((H, T * H), lambda i: (0, 0)),
            pl.BlockSpec((1, T * H), lambda i: (0, 0)),
        ],
        out_specs=pl.BlockSpec((blk, H), lambda i: (i, 0)),
        out_shape=jax.ShapeDtypeStruct((EH, H), jnp.float32),
    )(h_i32, et2d, w2cat, b2flat)


def _sc_scatter_body(m0_hbm, dst_hbm, zrows_hbm, out_hbm,
                     dstb, buf, agg_sh, sd, sm):
    c = lax.axis_index("c")
    s = lax.axis_index("s")
    wid = s * NC + c

    pltpu.sync_copy(zrows_hbm, agg_sh.at[pl.ds(s * NROWS, NROWS)])

    @pl.when(s == 0)
    def _():
        pltpu.sync_copy(zrows_hbm.at[pl.ds(0, NREM)],
                        agg_sh.at[pl.ds(NS * NROWS, NREM)])

    plsc.subcore_barrier()

    for m_hbm in (m0_hbm,):
        mbase0 = wid * EW
        dbase0 = wid * EW

        def issue(k, b):
            pltpu.async_copy(dst_hbm.at[pl.ds(dbase0 + k * GC, GC)],
                             dstb[b], sd[b])
            pltpu.async_copy(m_hbm.at[pl.ds(mbase0 + k * GC, GC)],
                             buf[b], sm[b])

        issue(0, 0)
        issue(1, 1)

        def step(k, b):
            pltpu.make_async_copy(dst_hbm.at[pl.ds(dbase0 + k * GC, GC)],
                                  dstb[b], sd[b]).wait()
            pltpu.make_async_copy(m_hbm.at[pl.ds(mbase0 + k * GC, GC)],
                                  buf[b], sm[b]).wait()
            pltpu.sync_copy(buf[b], agg_sh.at[dstb[b]], add=True)

            @pl.when(k < NCHUNK - 2)
            def _():
                issue(k + 2, b)

        def chunk(k, carry):
            @pl.when(k % 2 == 0)
            def _():
                step(k, 0)

            @pl.when(k % 2 == 1)
            def _():
                step(k, 1)

            return carry

        lax.fori_loop(0, NCHUNK, chunk, 0)

    plsc.subcore_barrier()

    pltpu.sync_copy(agg_sh.at[pl.ds(s * NROWS, NROWS)],
                    out_hbm.at[c].at[pl.ds(s * NROWS, NROWS)])

    @pl.when(s == 0)
    def _():
        pltpu.sync_copy(agg_sh.at[pl.ds(NS * NROWS, NREM)],
                        out_hbm.at[c].at[pl.ds(NS * NROWS, NREM)])


def _sc_scatter(m0, dst, zrows):
    mesh = plsc.VectorSubcoreMesh(core_axis_name="c", subcore_axis_name="s")
    f = pl.kernel(
        _sc_scatter_body,
        out_type=jax.ShapeDtypeStruct((NC, N, H), jnp.float32),
        mesh=mesh,
        scratch_types=[
            [pltpu.VMEM((GC,), jnp.int32), pltpu.VMEM((GC,), jnp.int32)],
            [pltpu.VMEM((GC, H), jnp.float32), pltpu.VMEM((GC, H), jnp.float32)],
            pltpu.VMEM_SHARED((N, H), jnp.float32),
            [pltpu.SemaphoreType.DMA, pltpu.SemaphoreType.DMA],
            [pltpu.SemaphoreType.DMA, pltpu.SemaphoreType.DMA],
        ],
    )
    return f(m0, dst, zrows)


def _k5_body(p_ref, x_ref, wih_ref, bih_ref, whh_ref, bhh_ref, out_ref):
    agg = p_ref[0] + p_ref[1]
    x = x_ref[...]
    gi = jnp.dot(agg, wih_ref[...], preferred_element_type=jnp.float32) + bih_ref[...]
    gh = jnp.dot(x, whh_ref[...], preferred_element_type=jnp.float32) + bhh_ref[...]
    r = jax.nn.sigmoid(gi[:, :H] + gh[:, :H])
    z = jax.nn.sigmoid(gi[:, H:2 * H] + gh[:, H:2 * H])
    n = jnp.tanh(gi[:, 2 * H:] + r * gh[:, 2 * H:])
    out_ref[...] = (1.0 - z) * n + z * x


def _gru(partials, x, w_ih, b_ih, w_hh, b_hh):
    blk = 1000
    return pl.pallas_call(
        _k5_body,
        grid=(N // blk,),
        in_specs=[
            pl.BlockSpec((NC, blk, H), lambda i: (0, i, 0)),
            pl.BlockSpec((blk, H), lambda i: (i, 0)),
            pl.BlockSpec((H, 3 * H), lambda i: (0, 0)),
            pl.BlockSpec((1, 3 * H), lambda i: (0, 0)),
            pl.BlockSpec((H, 3 * H), lambda i: (0, 0)),
            pl.BlockSpec((1, 3 * H), lambda i: (0, 0)),
        ],
        out_specs=pl.BlockSpec((blk, H), lambda i: (i, 0)),
        out_shape=jax.ShapeDtypeStruct((N, H), jnp.float32),
    )(partials, x, w_ih, b_ih, w_hh, b_hh)


def kernel(x, edge_index, edge_type, W1, b1, W2, b2, W_ih, b_ih, W_hh, b_hh):
    src = edge_index[0]
    dst = edge_index[1]

    wsrc = jnp.transpose(W1[:, :H, :], (1, 0, 2)).reshape(H, T * H)
    wdst = jnp.transpose(W1[:, H:, :], (1, 0, 2)).reshape(H, T * H)
    b1flat = b1.reshape(1, T * H)
    w2cat = jnp.transpose(W2, (1, 0, 2)).reshape(H, T * H)
    b2flat = b2.reshape(1, T * H)

    y1, y2 = _precompute_y(x, wsrc, wdst, b1flat)
    y1f = y1.reshape(N * T, H)
    y2f = y2.reshape(N * T, H)

    rows = E // H
    idx1_2d, idx2_2d = _edge_indices(
        src.reshape(rows, H), dst.reshape(rows, H), edge_type.reshape(rows, H))

    idx1 = idx1_2d.reshape(E)
    idx2 = idx2_2d.reshape(E)
    et2d = edge_type.reshape(E, 1)

    h_i32 = _sc_gather(y1f, y2f, idx1, idx2)
    w2p = w2cat[jnp.array(_PERM[0::2] + _PERM[1::2]), :]
    msgs = _second_layer(h_i32, et2d, w2p, b2flat)

    zrows = jnp.zeros((NROWS, H), jnp.float32)
    partials = _sc_scatter(msgs, dst, zrows)

    return _gru(partials, x, W_ih, b_ih.reshape(1, 3 * H), W_hh,
                b_hh.reshape(1, 3 * H))


# speedup vs baseline: 2.2086x; 1.0751x over previous
import functools

import jax
import jax.numpy as jnp
from jax import lax
from jax.experimental import pallas as pl
from jax.experimental.pallas import tpu as pltpu
from jax.experimental.pallas import tpu_sc as plsc

N = 10000
E = 320000
H = 128
T = 4

NC = 2
NS = 16
NW = NC * NS
EH = E
EW = EH // NW
GC = 80
NCHUNK = EW // GC
NROWS = 624
NREM = N - NS * NROWS

_PERM = []
for _g in range(4):
    for _i in range(16):
        _PERM.append(_g * 32 + _i)
        _PERM.append(_g * 32 + 16 + _i)


def _k1_body(x_ref, wsrc_ref, wdst_ref, b1_ref, y1_ref, y2_ref):
    x = x_ref[...]
    y1 = jnp.dot(x, wsrc_ref[...], preferred_element_type=jnp.float32)
    y2 = (jnp.dot(x, wdst_ref[...], preferred_element_type=jnp.float32)
          + b1_ref[...])
    y1_ref[...] = y1
    y2_ref[...] = y2


def _precompute_y(x, wsrc, wdst, b1flat):
    blk = 1000
    return pl.pallas_call(
        _k1_body,
        grid=(N // blk,),
        in_specs=[
            pl.BlockSpec((blk, H), lambda i: (i, 0)),
            pl.BlockSpec((H, T * H), lambda i: (0, 0)),
            pl.BlockSpec((H, T * H), lambda i: (0, 0)),
            pl.BlockSpec((1, T * H), lambda i: (0, 0)),
        ],
        out_specs=[
            pl.BlockSpec((blk, T * H), lambda i: (i, 0)),
            pl.BlockSpec((blk, T * H), lambda i: (i, 0)),
        ],
        out_shape=[
            jax.ShapeDtypeStruct((N, T * H), jnp.float32),
            jax.ShapeDtypeStruct((N, T * H), jnp.float32),
        ],
    )(x, wsrc, wdst, b1flat)


def _k1b_body(src_ref, dst_ref, et_ref, i1_ref, i2_ref):
    et = et_ref[...]
    i1_ref[...] = src_ref[...] * T + et
    i2_ref[...] = dst_ref[...] * T + et


def _edge_indices(src2d, dst2d, et2d):
    rows = E // H
    return pl.pallas_call(
        _k1b_body,
        grid=(1,),
        in_specs=[pl.BlockSpec((rows, H), lambda i: (0, 0))] * 3,
        out_specs=[pl.BlockSpec((rows, H), lambda i: (0, 0))] * 2,
        out_shape=[jax.ShapeDtypeStruct((rows, H), jnp.int32)] * 2,
    )(src2d, dst2d, et2d)


def _sc_gather_body(y1_hbm, y2_hbm, i1_hbm, i2_hbm, hm_hbm,
                    i1_all, i2_all, abuf, bbuf, obuf, sa, sb, so):
    c = lax.axis_index("c")
    s = lax.axis_index("s")
    wid = s * NC + c
    gbase = wid * EW
    base0 = wid * EW

    pltpu.sync_copy(i1_hbm.at[pl.ds(gbase, EW)], i1_all)
    pltpu.sync_copy(i2_hbm.at[pl.ds(gbase, EW)], i2_all)

    def issue(k, b):
        off = k * GC
        pltpu.async_copy(y1_hbm.at[i1_all.at[pl.ds(off, GC)]], abuf[b], sa[b])
        pltpu.async_copy(y2_hbm.at[i2_all.at[pl.ds(off, GC)]], bbuf[b], sb[b])

    issue(0, 0)
    issue(1, 1)

    def step(k, b):
        off = k * GC
        pltpu.make_async_copy(y1_hbm.at[i1_all.at[pl.ds(off, GC)]],
                              abuf[b], sa[b]).wait()
        pltpu.make_async_copy(y2_hbm.at[i2_all.at[pl.ds(off, GC)]],
                              bbuf[b], sb[b]).wait()

        @pl.when(k >= 2)
        def _():
            pltpu.make_async_copy(
                obuf[b], hm_hbm.at[pl.ds(base0 + (k - 2) * GC, GC)],
                so[b]).wait()

        def rows(r, carry):
            for g in range(H // 32):
                sl0 = pl.ds(g * 32, 16)
                sl1 = pl.ds(g * 32 + 16, 16)
                s0 = jnp.maximum(abuf[b][r, sl0] + bbuf[b][r, sl0], 0.0)
                s1 = jnp.maximum(abuf[b][r, sl1] + bbuf[b][r, sl1], 0.0)
                p = plsc.pack(s0, s1, format=plsc.PackFormat.INTERLEAVED)
                obuf[b][r, pl.ds(g * 16, 16)] = plsc.bitcast(p, jnp.int32)
            return carry

        lax.fori_loop(0, GC, rows, 0)

        @pl.when(k < NCHUNK - 2)
        def _():
            issue(k + 2, b)

        pltpu.async_copy(obuf[b], hm_hbm.at[pl.ds(base0 + off, GC)], so[b])

    def chunk(k, carry):
        @pl.when(k % 2 == 0)
        def _():
            step(k, 0)

        @pl.when(k % 2 == 1)
        def _():
            step(k, 1)

        return carry

    lax.fori_loop(0, NCHUNK, chunk, 0)

    for b, k in ((0, NCHUNK - 1), (1, NCHUNK - 2)):
        pltpu.make_async_copy(
            obuf[b], hm_hbm.at[pl.ds(base0 + k * GC, GC)], so[b]).wait()


def _sc_gather(y1f, y2f, idx1, idx2):
    mesh = plsc.VectorSubcoreMesh(core_axis_name="c", subcore_axis_name="s")
    vm = lambda: pltpu.VMEM((GC, H), jnp.float32)
    vo = lambda: pltpu.VMEM((GC, H // 2), jnp.int32)
    f = pl.kernel(
        _sc_gather_body,
        out_type=jax.ShapeDtypeStruct((EH, H // 2), jnp.int32),
        mesh=mesh,
        compiler_params=pltpu.CompilerParams(needs_layout_passes=False),
        scratch_types=[
            pltpu.VMEM((EW,), jnp.int32),
            pltpu.VMEM((EW,), jnp.int32),
            [vm(), vm()],
            [vm(), vm()],
            [vo(), vo()],
            [pltpu.SemaphoreType.DMA, pltpu.SemaphoreType.DMA],
            [pltpu.SemaphoreType.DMA, pltpu.SemaphoreType.DMA],
            [pltpu.SemaphoreType.DMA, pltpu.SemaphoreType.DMA],
        ],
    )
    return f(y1f, y2f, idx1, idx2)


def _k3_body(h_ref, et_ref, w2_ref, b2_ref, out_ref):
    w = h_ref[...]
    hlo = jax.lax.bitcast_convert_type(jnp.left_shift(w, 16), jnp.float32)
    hhi = jax.lax.bitcast_convert_type(
        jnp.bitwise_and(w, jnp.int32(-65536)), jnp.float32)
    h = jnp.concatenate([hlo, hhi], axis=1)
    m = jnp.dot(h, w2_ref[...], preferred_element_type=jnp.float32)
    et = et_ref[...]
    b0 = jnp.bitwise_and(et, 1) == 1
    b1 = jnp.bitwise_and(et, 2) == 2
    m01 = jnp.where(b0, m[:, H:2 * H], m[:, :H])
    m23 = jnp.where(b0, m[:, 3 * H:], m[:, 2 * H:3 * H])
    sel = jnp.where(b1, m23, m01)
    b2 = b2_ref[...]
    b01 = jnp.where(b0, b2[0:1, H:2 * H], b2[0:1, :H])
    b23 = jnp.where(b0, b2[0:1, 3 * H:], b2[0:1, 2 * H:3 * H])
    out_ref[...] = sel + jnp.where(b1, b23, b01)


def _second_layer(h_i32, et2d, w2cat, b2flat):
    blk = 4000
    return pl.pallas_call(
        _k3_body,
        grid=(EH // blk,),
        in_specs=[
            pl.BlockSpec((blk, H // 2), lambda i: (i, 0)),
            pl.BlockSpec((blk, 1), lambda i: (i, 0)),
            pl.BlockSpec((H, T * H), lambda i: (0, 0)),
            pl.BlockSpec((1, T * H), lambda i: (0, 0)),
        ],
        out_specs=pl.BlockSpec((blk, H), lambda i: (i, 0)),
        out_shape=jax.ShapeDtypeStruct((EH, H), jnp.float32),
    )(h_i32, et2d, w2cat, b2flat)


def _sc_scatter_body(m0_hbm, dst_hbm, zrows_hbm, out_hbm,
                     dstb, buf, agg_sh, sd, sm):
    c = lax.axis_index("c")
    s = lax.axis_index("s")
    wid = s * NC + c

    pltpu.sync_copy(zrows_hbm, agg_sh.at[pl.ds(s * NROWS, NROWS)])

    @pl.when(s == 0)
    def _():
        pltpu.sync_copy(zrows_hbm.at[pl.ds(0, NREM)],
                        agg_sh.at[pl.ds(NS * NROWS, NREM)])

    plsc.subcore_barrier()

    for m_hbm in (m0_hbm,):
        mbase0 = wid * EW
        dbase0 = wid * EW

        def issue(k, b):
            pltpu.async_copy(dst_hbm.at[pl.ds(dbase0 + k * GC, GC)],
                             dstb[b], sd[b])
            pltpu.async_copy(m_hbm.at[pl.ds(mbase0 + k * GC, GC)],
                             buf[b], sm[b])

        issue(0, 0)
        issue(1, 1)

        def step(k, b):
            pltpu.make_async_copy(dst_hbm.at[pl.ds(dbase0 + k * GC, GC)],
                                  dstb[b], sd[b]).wait()
            pltpu.make_async_copy(m_hbm.at[pl.ds(mbase0 + k * GC, GC)],
                                  buf[b], sm[b]).wait()
            pltpu.sync_copy(buf[b], agg_sh.at[dstb[b]], add=True)

            @pl.when(k < NCHUNK - 2)
            def _():
                issue(k + 2, b)

        def chunk(k, carry):
            @pl.when(k % 2 == 0)
            def _():
                step(k, 0)

            @pl.when(k % 2 == 1)
            def _():
                step(k, 1)

            return carry

        lax.fori_loop(0, NCHUNK, chunk, 0)

    plsc.subcore_barrier()

    pltpu.sync_copy(agg_sh.at[pl.ds(s * NROWS, NROWS)],
                    out_hbm.at[c].at[pl.ds(s * NROWS, NROWS)])

    @pl.when(s == 0)
    def _():
        pltpu.sync_copy(agg_sh.at[pl.ds(NS * NROWS, NREM)],
                        out_hbm.at[c].at[pl.ds(NS * NROWS, NREM)])


def _sc_scatter(m0, dst, zrows):
    mesh = plsc.VectorSubcoreMesh(core_axis_name="c", subcore_axis_name="s")
    f = pl.kernel(
        _sc_scatter_body,
        out_type=jax.ShapeDtypeStruct((NC, N, H), jnp.float32),
        mesh=mesh,
        scratch_types=[
            [pltpu.VMEM((GC,), jnp.int32), pltpu.VMEM((GC,), jnp.int32)],
            [pltpu.VMEM((GC, H), jnp.float32), pltpu.VMEM((GC, H), jnp.float32)],
            pltpu.VMEM_SHARED((N, H), jnp.float32),
            [pltpu.SemaphoreType.DMA, pltpu.SemaphoreType.DMA],
            [pltpu.SemaphoreType.DMA, pltpu.SemaphoreType.DMA],
        ],
    )
    return f(m0, dst, zrows)


def _k5_body(p_ref, x_ref, wih_ref, bih_ref, whh_ref, bhh_ref, out_ref):
    agg = p_ref[0] + p_ref[1]
    x = x_ref[...]
    gi = jnp.dot(agg, wih_ref[...], preferred_element_type=jnp.float32) + bih_ref[...]
    gh = jnp.dot(x, whh_ref[...], preferred_element_type=jnp.float32) + bhh_ref[...]
    r = jax.nn.sigmoid(gi[:, :H] + gh[:, :H])
    z = jax.nn.sigmoid(gi[:, H:2 * H] + gh[:, H:2 * H])
    n = jnp.tanh(gi[:, 2 * H:] + r * gh[:, 2 * H:])
    out_ref[...] = (1.0 - z) * n + z * x


def _gru(partials, x, w_ih, b_ih, w_hh, b_hh):
    blk = 1000
    return pl.pallas_call(
        _k5_body,
        grid=(N // blk,),
        in_specs=[
            pl.BlockSpec((NC, blk, H), lambda i: (0, i, 0)),
            pl.BlockSpec((blk, H), lambda i: (i, 0)),
            pl.BlockSpec((H, 3 * H), lambda i: (0, 0)),
            pl.BlockSpec((1, 3 * H), lambda i: (0, 0)),
            pl.BlockSpec((H, 3 * H), lambda i: (0, 0)),
            pl.BlockSpec((1, 3 * H), lambda i: (0, 0)),
        ],
        out_specs=pl.BlockSpec((blk, H), lambda i: (i, 0)),
        out_shape=jax.ShapeDtypeStruct((N, H), jnp.float32),
    )(partials, x, w_ih, b_ih, w_hh, b_hh)


def kernel(x, edge_index, edge_type, W1, b1, W2, b2, W_ih, b_ih, W_hh, b_hh):
    src = edge_index[0]
    dst = edge_index[1]

    wsrc = jnp.transpose(W1[:, :H, :], (1, 0, 2)).reshape(H, T * H)
    wdst = jnp.transpose(W1[:, H:, :], (1, 0, 2)).reshape(H, T * H)
    b1flat = b1.reshape(1, T * H)
    w2cat = jnp.transpose(W2, (1, 0, 2)).reshape(H, T * H)
    b2flat = b2.reshape(1, T * H)

    y1, y2 = _precompute_y(x, wsrc, wdst, b1flat)
    y1f = y1.reshape(N * T, H)
    y2f = y2.reshape(N * T, H)

    rows = E // H
    idx1_2d, idx2_2d = _edge_indices(
        src.reshape(rows, H), dst.reshape(rows, H), edge_type.reshape(rows, H))

    idx1 = idx1_2d.reshape(E)
    idx2 = idx2_2d.reshape(E)
    et2d = edge_type.reshape(E, 1)

    h_i32 = _sc_gather(y1f, y2f, idx1, idx2)
    w2p = w2cat[jnp.array(_PERM[0::2] + _PERM[1::2]), :]
    msgs = _second_layer(h_i32, et2d, w2p, b2flat)

    zrows = jnp.zeros((NROWS, H), jnp.float32)
    partials = _sc_scatter(msgs, dst, zrows)

    return _gru(partials, x, W_ih, b_ih.reshape(1, 3 * H), W_hh,
                b_hh.reshape(1, 3 * H))


# speedup vs baseline: 2.2837x; 1.0340x over previous
import functools

import jax
import jax.numpy as jnp
from jax import lax
from jax.experimental import pallas as pl
from jax.experimental.pallas import tpu as pltpu
from jax.experimental.pallas import tpu_sc as plsc

N = 10000
E = 320000
H = 128
T = 4

NC = 2
NS = 16
NW = NC * NS
EH = E
EW = EH // NW
GC = 80
NCHUNK = EW // GC
NROWS = 624
NREM = N - NS * NROWS

_PERM = []
for _g in range(4):
    for _i in range(16):
        _PERM.append(_g * 32 + _i)
        _PERM.append(_g * 32 + 16 + _i)


def _k1_body(x_ref, wsrc_ref, wdst_ref, b1_ref, y1_ref, y2_ref):
    x = x_ref[...]
    y1 = jnp.dot(x, wsrc_ref[...], preferred_element_type=jnp.float32)
    y2 = (jnp.dot(x, wdst_ref[...], preferred_element_type=jnp.float32)
          + b1_ref[...])
    y1_ref[...] = y1
    y2_ref[...] = y2


def _precompute_y(x, wsrc, wdst, b1flat):
    blk = 1000
    return pl.pallas_call(
        _k1_body,
        grid=(N // blk,),
        in_specs=[
            pl.BlockSpec((blk, H), lambda i: (i, 0)),
            pl.BlockSpec((H, T * H), lambda i: (0, 0)),
            pl.BlockSpec((H, T * H), lambda i: (0, 0)),
            pl.BlockSpec((1, T * H), lambda i: (0, 0)),
        ],
        out_specs=[
            pl.BlockSpec((blk, T * H), lambda i: (i, 0)),
            pl.BlockSpec((blk, T * H), lambda i: (i, 0)),
        ],
        out_shape=[
            jax.ShapeDtypeStruct((N, T * H), jnp.float32),
            jax.ShapeDtypeStruct((N, T * H), jnp.float32),
        ],
    )(x, wsrc, wdst, b1flat)


def _k1b_body(src_ref, dst_ref, et_ref, i1_ref, i2_ref):
    et = et_ref[...]
    i1_ref[...] = src_ref[...] * T + et
    i2_ref[...] = dst_ref[...] * T + et


def _edge_indices(src2d, dst2d, et2d):
    rows = E // H
    return pl.pallas_call(
        _k1b_body,
        grid=(1,),
        in_specs=[pl.BlockSpec((rows, H), lambda i: (0, 0))] * 3,
        out_specs=[pl.BlockSpec((rows, H), lambda i: (0, 0))] * 2,
        out_shape=[jax.ShapeDtypeStruct((rows, H), jnp.int32)] * 2,
    )(src2d, dst2d, et2d)


def _sc_gather_body(y1_hbm, y2_hbm, i1_hbm, i2_hbm, hm_hbm,
                    i1_all, i2_all, abuf, bbuf, obuf, sa, sb, so):
    c = lax.axis_index("c")
    s = lax.axis_index("s")
    wid = s * NC + c
    gbase = wid * EW
    base0 = wid * EW

    pltpu.sync_copy(i1_hbm.at[pl.ds(gbase, EW)], i1_all)
    pltpu.sync_copy(i2_hbm.at[pl.ds(gbase, EW)], i2_all)

    def issue(k, b):
        off = k * GC
        pltpu.async_copy(y1_hbm.at[i1_all.at[pl.ds(off, GC)]], abuf[b], sa[b])
        pltpu.async_copy(y2_hbm.at[i2_all.at[pl.ds(off, GC)]], bbuf[b], sb[b])

    issue(0, 0)
    issue(1, 1)

    def step(k, b):
        off = k * GC
        pltpu.make_async_copy(y1_hbm.at[i1_all.at[pl.ds(off, GC)]],
                              abuf[b], sa[b]).wait()
        pltpu.make_async_copy(y2_hbm.at[i2_all.at[pl.ds(off, GC)]],
                              bbuf[b], sb[b]).wait()

        @pl.when(k >= 2)
        def _():
            pltpu.make_async_copy(
                obuf[b], hm_hbm.at[pl.ds(base0 + (k - 2) * GC, GC)],
                so[b]).wait()

        def rows(r, carry):
            for g in range(H // 32):
                sl0 = pl.ds(g * 32, 16)
                sl1 = pl.ds(g * 32 + 16, 16)
                s0 = jnp.maximum(abuf[b][r, sl0] + bbuf[b][r, sl0], 0.0)
                s1 = jnp.maximum(abuf[b][r, sl1] + bbuf[b][r, sl1], 0.0)
                p = plsc.pack(s0, s1, format=plsc.PackFormat.INTERLEAVED)
                obuf[b][r, pl.ds(g * 16, 16)] = plsc.bitcast(p, jnp.int32)
            return carry

        lax.fori_loop(0, GC, rows, 0)

        @pl.when(k < NCHUNK - 2)
        def _():
            issue(k + 2, b)

        pltpu.async_copy(obuf[b], hm_hbm.at[pl.ds(base0 + off, GC)], so[b])

    def chunk(k, carry):
        @pl.when(k % 2 == 0)
        def _():
            step(k, 0)

        @pl.when(k % 2 == 1)
        def _():
            step(k, 1)

        return carry

    lax.fori_loop(0, NCHUNK, chunk, 0)

    for b, k in ((0, NCHUNK - 1), (1, NCHUNK - 2)):
        pltpu.make_async_copy(
            obuf[b], hm_hbm.at[pl.ds(base0 + k * GC, GC)], so[b]).wait()


def _sc_gather(y1f, y2f, idx1, idx2):
    mesh = plsc.VectorSubcoreMesh(core_axis_name="c", subcore_axis_name="s")
    vm = lambda: pltpu.VMEM((GC, H), jnp.float32)
    vo = lambda: pltpu.VMEM((GC, H // 2), jnp.int32)
    f = pl.kernel(
        _sc_gather_body,
        out_type=jax.ShapeDtypeStruct((EH, H // 2), jnp.int32),
        mesh=mesh,
        compiler_params=pltpu.CompilerParams(needs_layout_passes=False),
        scratch_types=[
            pltpu.VMEM((EW,), jnp.int32),
            pltpu.VMEM((EW,), jnp.int32),
            [vm(), vm()],
            [vm(), vm()],
            [vo(), vo()],
            [pltpu.SemaphoreType.DMA, pltpu.SemaphoreType.DMA],
            [pltpu.SemaphoreType.DMA, pltpu.SemaphoreType.DMA],
            [pltpu.SemaphoreType.DMA, pltpu.SemaphoreType.DMA],
        ],
    )
    return f(y1f, y2f, idx1, idx2)


def _k3_body(h_ref, et_ref, w2_ref, b2_ref, out_ref):
    w = h_ref[...]
    hlo = jax.lax.bitcast_convert_type(jnp.left_shift(w, 16), jnp.float32)
    hhi = jax.lax.bitcast_convert_type(
        jnp.bitwise_and(w, jnp.int32(-65536)), jnp.float32)
    h = jnp.concatenate([hlo, hhi], axis=1)
    m = jnp.dot(h, w2_ref[...], preferred_element_type=jnp.float32)
    et = et_ref[...]
    b0 = jnp.bitwise_and(et, 1) == 1
    b1 = jnp.bitwise_and(et, 2) == 2
    m01 = jnp.where(b0, m[:, H:2 * H], m[:, :H])
    m23 = jnp.where(b0, m[:, 3 * H:], m[:, 2 * H:3 * H])
    sel = jnp.where(b1, m23, m01)
    b2 = b2_ref[...]
    b01 = jnp.where(b0, b2[0:1, H:2 * H], b2[0:1, :H])
    b23 = jnp.where(b0, b2[0:1, 3 * H:], b2[0:1, 2 * H:3 * H])
    out_ref[...] = sel + jnp.where(b1, b23, b01)


def _second_layer(h_i32, et2d, w2cat, b2flat):
    blk = 8000
    return pl.pallas_call(
        _k3_body,
        grid=(EH // blk,),
        in_specs=[
            pl.BlockSpec((blk, H // 2), lambda i: (i, 0)),
            pl.BlockSpec((blk, 1), lambda i: (i, 0)),
            pl.BlockSpec((H, T * H), lambda i: (0, 0)),
            pl.BlockSpec((1, T * H), lambda i: (0, 0)),
        ],
        out_specs=pl.BlockSpec((blk, H), lambda i: (i, 0)),
        out_shape=jax.ShapeDtypeStruct((EH, H), jnp.float32),
    )(h_i32, et2d, w2cat, b2flat)


def _sc_scatter_body(m0_hbm, dst_hbm, zrows_hbm, out_hbm,
                     dstb, buf, agg_sh, sd, sm):
    c = lax.axis_index("c")
    s = lax.axis_index("s")
    wid = s * NC + c

    pltpu.sync_copy(zrows_hbm, agg_sh.at[pl.ds(s * NROWS, NROWS)])

    @pl.when(s == 0)
    def _():
        pltpu.sync_copy(zrows_hbm.at[pl.ds(0, NREM)],
                        agg_sh.at[pl.ds(NS * NROWS, NREM)])

    plsc.subcore_barrier()

    for m_hbm in (m0_hbm,):
        mbase0 = wid * EW
        dbase0 = wid * EW

        def issue(k, b):
            pltpu.async_copy(dst_hbm.at[pl.ds(dbase0 + k * GC, GC)],
                             dstb[b], sd[b])
            pltpu.async_copy(m_hbm.at[pl.ds(mbase0 + k * GC, GC)],
                             buf[b], sm[b])

        issue(0, 0)
        issue(1, 1)

        def step(k, b):
            pltpu.make_async_copy(dst_hbm.at[pl.ds(dbase0 + k * GC, GC)],
                                  dstb[b], sd[b]).wait()
            pltpu.make_async_copy(m_hbm.at[pl.ds(mbase0 + k * GC, GC)],
                                  buf[b], sm[b]).wait()
            pltpu.sync_copy(buf[b], agg_sh.at[dstb[b]], add=True)

            @pl.when(k < NCHUNK - 2)
            def _():
                issue(k + 2, b)

        def chunk(k, carry):
            @pl.when(k % 2 == 0)
            def _():
                step(k, 0)

            @pl.when(k % 2 == 1)
            def _():
                step(k, 1)

            return carry

        lax.fori_loop(0, NCHUNK, chunk, 0)

    plsc.subcore_barrier()

    pltpu.sync_copy(agg_sh.at[pl.ds(s * NROWS, NROWS)],
                    out_hbm.at[c].at[pl.ds(s * NROWS, NROWS)])

    @pl.when(s == 0)
    def _():
        pltpu.sync_copy(agg_sh.at[pl.ds(NS * NROWS, NREM)],
                        out_hbm.at[c].at[pl.ds(NS * NROWS, NREM)])


def _sc_scatter(m0, dst, zrows):
    mesh = plsc.VectorSubcoreMesh(core_axis_name="c", subcore_axis_name="s")
    f = pl.kernel(
        _sc_scatter_body,
        out_type=jax.ShapeDtypeStruct((NC, N, H), jnp.float32),
        mesh=mesh,
        scratch_types=[
            [pltpu.VMEM((GC,), jnp.int32), pltpu.VMEM((GC,), jnp.int32)],
            [pltpu.VMEM((GC, H), jnp.float32), pltpu.VMEM((GC, H), jnp.float32)],
            pltpu.VMEM_SHARED((N, H), jnp.float32),
            [pltpu.SemaphoreType.DMA, pltpu.SemaphoreType.DMA],
            [pltpu.SemaphoreType.DMA, pltpu.SemaphoreType.DMA],
        ],
    )
    return f(m0, dst, zrows)


def _k5_body(p_ref, x_ref, wih_ref, bih_ref, whh_ref, bhh_ref, out_ref):
    agg = p_ref[0] + p_ref[1]
    x = x_ref[...]
    gi = jnp.dot(agg, wih_ref[...], preferred_element_type=jnp.float32) + bih_ref[...]
    gh = jnp.dot(x, whh_ref[...], preferred_element_type=jnp.float32) + bhh_ref[...]
    r = jax.nn.sigmoid(gi[:, :H] + gh[:, :H])
    z = jax.nn.sigmoid(gi[:, H:2 * H] + gh[:, H:2 * H])
    n = jnp.tanh(gi[:, 2 * H:] + r * gh[:, 2 * H:])
    out_ref[...] = (1.0 - z) * n + z * x


def _gru(partials, x, w_ih, b_ih, w_hh, b_hh):
    blk = 1000
    return pl.pallas_call(
        _k5_body,
        grid=(N // blk,),
        in_specs=[
            pl.BlockSpec((NC, blk, H), lambda i: (0, i, 0)),
            pl.BlockSpec((blk, H), lambda i: (i, 0)),
            pl.BlockSpec((H, 3 * H), lambda i: (0, 0)),
            pl.BlockSpec((1, 3 * H), lambda i: (0, 0)),
            pl.BlockSpec((H, 3 * H), lambda i: (0, 0)),
            pl.BlockSpec((1, 3 * H), lambda i: (0, 0)),
        ],
        out_specs=pl.BlockSpec((blk, H), lambda i: (i, 0)),
        out_shape=jax.ShapeDtypeStruct((N, H), jnp.float32),
    )(partials, x, w_ih, b_ih, w_hh, b_hh)


def kernel(x, edge_index, edge_type, W1, b1, W2, b2, W_ih, b_ih, W_hh, b_hh):
    src = edge_index[0]
    dst = edge_index[1]

    wsrc = jnp.transpose(W1[:, :H, :], (1, 0, 2)).reshape(H, T * H)
    wdst = jnp.transpose(W1[:, H:, :], (1, 0, 2)).reshape(H, T * H)
    b1flat = b1.reshape(1, T * H)
    w2cat = jnp.transpose(W2, (1, 0, 2)).reshape(H, T * H)
    b2flat = b2.reshape(1, T * H)

    y1, y2 = _precompute_y(x, wsrc, wdst, b1flat)
    y1f = y1.reshape(N * T, H)
    y2f = y2.reshape(N * T, H)

    rows = E // H
    idx1_2d, idx2_2d = _edge_indices(
        src.reshape(rows, H), dst.reshape(rows, H), edge_type.reshape(rows, H))

    idx1 = idx1_2d.reshape(E)
    idx2 = idx2_2d.reshape(E)
    et2d = edge_type.reshape(E, 1)

    h_i32 = _sc_gather(y1f, y2f, idx1, idx2)
    w2p = w2cat[jnp.array(_PERM[0::2] + _PERM[1::2]), :]
    msgs = _second_layer(h_i32, et2d, w2p, b2flat)

    zrows = jnp.zeros((NROWS, H), jnp.float32)
    partials = _sc_scatter(msgs, dst, zrows)

    return _gru(partials, x, W_ih, b_ih.reshape(1, 3 * H), W_hh,
                b_hh.reshape(1, 3 * H))
